# baseline (speedup 1.0000x reference)
import jax
import jax.numpy as jnp
from jax import lax
from jax.experimental import pallas as pl
from jax.experimental.pallas import tpu as pltpu
from jax.experimental.pallas import tpu_sc as plsc

N = 10000
F = 128
HF = 64
G = 64
C = 10

NC = 2
NS = 16
NW = NC * NS

N_PAD = 10240
ROWS_PER_TILE = N_PAD // NS
ABSORB = N

K = 128
NBUF = 3

_MESH = plsc.VectorSubcoreMesh(
    core_axis_name="c", subcore_axis_name="s", num_cores=NC, num_subcores=NS)


def _sc_body_seg(m0, m1, args):
    (x0_hbm, x1_hbm, src_hbm, dst_hbm, zrow_hbm, z16_hbm, ones_hbm,
     seg0_hbm, seg1_hbm, cnt_hbm, src_all, dst_all, g0, g1, g2,
     sg0, sg1, sg2, ss0, ss1, ss2,
     accum, onesv, c16v, cntacc, sem_c) = args
    gbufs = (g0, g1, g2)
    sem_g = (sg0, sg1, sg2)
    sem_s = (ss0, ss1, ss2)

    cid = lax.axis_index("c")
    sid = lax.axis_index("s")
    row0 = sid * ROWS_PER_TILE
    rows_sl = pl.ds(row0, ROWS_PER_TILE)

    m = jnp.where(cid == 0, m0, m1)
    base = jnp.where(cid == 0, sid * m0, NS * m0 + sid * m1)

    pltpu.sync_copy(src_hbm.at[pl.ds(base, m1)],
                    src_all.at[pl.ds(0, m1)])
    pltpu.sync_copy(dst_hbm.at[pl.ds(base, m1)],
                    dst_all.at[pl.ds(0, m1)])

    @pl.when(cid == 0)
    def _():
        pltpu.sync_copy(src_hbm.at[pl.ds(base + m1, m0 - m1)],
                        src_all.at[pl.ds(m1, m0 - m1)])
        pltpu.sync_copy(dst_hbm.at[pl.ds(base + m1, m0 - m1)],
                        dst_all.at[pl.ds(m1, m0 - m1)])

    for p, (x_hbm, seg_hbm) in enumerate(((x0_hbm, seg0_hbm),
                                          (x1_hbm, seg1_hbm))):
        pltpu.sync_copy(zrow_hbm, g1)
        for z in range(ROWS_PER_TILE // K):
            pltpu.async_copy(g1, accum.at[pl.ds(row0 + z * K, K)],
                             sem_s[z % NBUF])
        if p == 0:
            pltpu.sync_copy(ones_hbm, onesv)
            pltpu.sync_copy(z16_hbm, c16v)
            pltpu.async_copy(c16v, cntacc.at[rows_sl], sem_c)
        for z in range(ROWS_PER_TILE // K):
            pltpu.make_async_copy(g1, accum.at[pl.ds(row0 + z * K, K)],
                                  sem_s[z % NBUF]).wait()
        if p == 0:
            pltpu.make_async_copy(c16v, cntacc.at[rows_sl], sem_c).wait()
        plsc.subcore_barrier()

        for b in range(NBUF - 1):
            pltpu.async_copy(x_hbm.at[src_all.at[b]], gbufs[b], sem_g[b])

        def iter_body(j, carry):
            for bb in range(NBUF):
                @pl.when(j % NBUF == bb)
                def _():
                    b2 = (bb + NBUF - 1) % NBUF
                    pltpu.make_async_copy(
                        x_hbm.at[src_all.at[0]], gbufs[bb], sem_g[bb]).wait()
                    pltpu.async_copy(gbufs[bb], accum.at[dst_all.at[j]],
                                     sem_s[bb], add=True)
                    if p == 0:
                        @pl.when(j >= 1)
                        def _():
                            pltpu.make_async_copy(
                                onesv, cntacc.at[dst_all.at[0]], sem_c).wait()
                        pltpu.async_copy(onesv, cntacc.at[dst_all.at[j]],
                                         sem_c, add=True)

                    @pl.when(j + NBUF - 1 < m)
                    def _():
                        @pl.when(j >= 1)
                        def _():
                            pltpu.make_async_copy(
                                gbufs[b2], accum.at[dst_all.at[0]],
                                sem_s[b2]).wait()
                        pltpu.async_copy(x_hbm.at[src_all.at[j + NBUF - 1]],
                                         gbufs[b2], sem_g[b2])
            return carry

        lax.fori_loop(0, m, iter_body, 0)
        for bb in range(NBUF):
            pltpu.make_async_copy(gbufs[bb], accum.at[dst_all.at[0]],
                                  sem_s[bb]).wait()
        if p == 0:
            pltpu.make_async_copy(onesv, cntacc.at[dst_all.at[0]],
                                  sem_c).wait()
        plsc.subcore_barrier()

        nz = ROWS_PER_TILE // K
        sls = [pl.ds(row0 + z * K, K) for z in range(nz)]
        if p == 0:
            pltpu.async_copy(cntacc.at[rows_sl], c16v, sem_c)
        for z in range(min(NBUF, nz)):
            pltpu.async_copy(accum.at[sls[z]], gbufs[z], sem_g[z])
        for z in range(nz):
            b = z % NBUF
            pltpu.make_async_copy(accum.at[sls[z]], gbufs[b],
                                  sem_g[b]).wait()
            pltpu.async_copy(gbufs[b], seg_hbm.at[cid].at[sls[z]], sem_s[b])
            if z + NBUF < nz:
                pltpu.make_async_copy(gbufs[b], seg_hbm.at[cid].at[sls[z]],
                                      sem_s[b]).wait()
                pltpu.async_copy(accum.at[sls[z + NBUF]], gbufs[b], sem_g[b])
        for z in range(max(0, nz - NBUF), nz):
            b = z % NBUF
            pltpu.make_async_copy(gbufs[b], seg_hbm.at[cid].at[sls[z]],
                                  sem_s[b]).wait()
        if p == 0:
            pltpu.make_async_copy(cntacc.at[rows_sl], c16v, sem_c).wait()
            pltpu.sync_copy(c16v, cnt_hbm.at[cid].at[rows_sl])
        plsc.subcore_barrier()


def _make_sc_aggregate(m0, m1):
    seg_t = jax.ShapeDtypeStruct((NC, N_PAD, HF), jnp.float32)
    out_type = (seg_t, seg_t, jax.ShapeDtypeStruct((NC, N_PAD, 16),
                                                   jnp.float32))
    scratch = [
        pltpu.VMEM((m0, K), jnp.int32),
        pltpu.VMEM((m0, K), jnp.int32),
        pltpu.VMEM((K, HF), jnp.float32),
        pltpu.VMEM((K, HF), jnp.float32),
        pltpu.VMEM((K, HF), jnp.float32),
        pltpu.SemaphoreType.DMA,
        pltpu.SemaphoreType.DMA,
        pltpu.SemaphoreType.DMA,
        pltpu.SemaphoreType.DMA,
        pltpu.SemaphoreType.DMA,
        pltpu.SemaphoreType.DMA,
        pltpu.VMEM_SHARED((N_PAD, HF), jnp.float32),
        pltpu.VMEM((K, 16), jnp.float32),
        pltpu.VMEM((ROWS_PER_TILE, 16), jnp.float32),
        pltpu.VMEM_SHARED((N_PAD, 16), jnp.float32),
        pltpu.SemaphoreType.DMA,
    ]

    def body(*args):
        _sc_body_seg(m0, m1, args)

    return pl.kernel(body, out_type=out_type, mesh=_MESH,
                     scratch_types=tuple(scratch),
                     compiler_params=pltpu.CompilerParams(
                         use_tc_tiling_on_sc=False))


ROW_BLK = 256
N_BLKS = N_PAD // ROW_BLK


def _tc_layer_body(seg0_ref, seg1_ref, cnt_ref, x0_ref, x1_ref, wl_ref,
                   bl_ref, wr_ref, o0_ref, o1_ref):
    seg = jnp.concatenate([seg0_ref[0] + seg0_ref[1],
                           seg1_ref[0] + seg1_ref[1]], axis=1)
    x = jnp.concatenate([x0_ref[...], x1_ref[...]], axis=1)
    cnt = jnp.sum(cnt_ref[0] + cnt_ref[1], axis=1, keepdims=True) * (1.0 / 16.0)
    mean = seg / jnp.maximum(cnt, 1.0)
    dn = (((1,), (1,)), ((), ()))
    h = (lax.dot_general(mean, wl_ref[...], dn,
                         preferred_element_type=jnp.float32)
         + bl_ref[0]
         + lax.dot_general(x, wr_ref[...], dn,
                           preferred_element_type=jnp.float32))
    h = jnp.maximum(h, 0.0)
    o0_ref[...] = h[:, :HF]
    o1_ref[...] = h[:, HF:]


_half_t = jax.ShapeDtypeStruct((N_PAD, HF), jnp.float32)

_tc_layer = pl.pallas_call(
    _tc_layer_body,
    grid=(N_BLKS,),
    in_specs=[
        pl.BlockSpec((NC, ROW_BLK, HF), lambda i: (0, i, 0)),
        pl.BlockSpec((NC, ROW_BLK, HF), lambda i: (0, i, 0)),
        pl.BlockSpec((NC, ROW_BLK, 16), lambda i: (0, i, 0)),
        pl.BlockSpec((ROW_BLK, HF), lambda i: (i, 0)),
        pl.BlockSpec((ROW_BLK, HF), lambda i: (i, 0)),
        pl.BlockSpec((F, F), lambda i: (0, 0)),
        pl.BlockSpec((1, F), lambda i: (0, 0)),
        pl.BlockSpec((F, F), lambda i: (0, 0)),
    ],
    out_specs=[pl.BlockSpec((ROW_BLK, HF), lambda i: (i, 0)),
               pl.BlockSpec((ROW_BLK, HF), lambda i: (i, 0))],
    out_shape=[_half_t, _half_t],
)


def _tc_pool_body(h0_ref, h1_ref, b_ref, fc1w_ref, fc1b_ref, fc2w_ref,
                  fc2b_ref, o_ref, sum_scr, cnt_scr):
    i = pl.program_id(0)
    bids = b_ref[0, 0, :]
    h = jnp.concatenate([h0_ref[...], h1_ref[...]], axis=1)
    onehot = (lax.broadcasted_iota(jnp.int32, (G, ROW_BLK), 0)
              == bids[None, :]).astype(jnp.float32)
    psum = jnp.dot(onehot, h, preferred_element_type=jnp.float32)
    pcnt = jnp.broadcast_to(jnp.sum(onehot, axis=1, keepdims=True), (G, F))

    @pl.when(i == 0)
    def _():
        sum_scr[...] = jnp.zeros_like(sum_scr)
        cnt_scr[...] = jnp.zeros_like(cnt_scr)

    sum_scr[...] += psum
    cnt_scr[...] += pcnt

    @pl.when(i == N_BLKS - 1)
    def _():
        pooled = sum_scr[...] / jnp.maximum(cnt_scr[...], 1.0)
        dn = (((1,), (1,)), ((), ()))
        emb = lax.dot_general(pooled, fc1w_ref[...], dn,
                              preferred_element_type=jnp.float32) + fc1b_ref[0]
        o_ref[...] = lax.dot_general(emb, fc2w_ref[...], dn,
                                     preferred_element_type=jnp.float32) + fc2b_ref[0]


_tc_pool = pl.pallas_call(
    _tc_pool_body,
    grid=(N_BLKS,),
    in_specs=[
        pl.BlockSpec((ROW_BLK, HF), lambda i: (i, 0)),
        pl.BlockSpec((ROW_BLK, HF), lambda i: (i, 0)),
        pl.BlockSpec((1, 1, ROW_BLK), lambda i: (i, 0, 0)),
        pl.BlockSpec((F, F), lambda i: (0, 0)),
        pl.BlockSpec((1, F), lambda i: (0, 0)),
        pl.BlockSpec((F, F), lambda i: (0, 0)),
        pl.BlockSpec((1, F), lambda i: (0, 0)),
    ],
    out_specs=pl.BlockSpec((G, F), lambda i: (0, 0)),
    out_shape=jax.ShapeDtypeStruct((G, F), jnp.float32),
    scratch_shapes=[pltpu.VMEM((G, F), jnp.float32),
                    pltpu.VMEM((G, F), jnp.float32)],
)


def kernel(x, edge_index, batch, num_graphs, Wl1, bl1, Wr1, Wl2, bl2, Wr2,
           Wl3, bl3, Wr3, Wl4, bl4, Wr4, fc1_W, fc1_b, fc2_W, fc2_b):
    E = edge_index.shape[1]
    ct = -(-E // K)
    frac = 0.89
    m0 = max(NBUF, int(frac * ct / NS) // NBUF * NBUF)
    m1 = max(NBUF, -(-max(0, ct - NS * m0) // (NS * NBUF)) * NBUF)
    rows = NS * (m0 + m1)

    src = jnp.pad(edge_index[0], (0, rows * K - E)).reshape(rows, K)
    pad_dst = ABSORB + (jnp.arange(rows * K - E, dtype=jnp.int32)
                        % (N_PAD - N))
    dst = jnp.concatenate([edge_index[1], pad_dst]).reshape(rows, K)

    x_pad = jnp.pad(x, ((0, N_PAD - N), (0, 0)))
    x0 = x_pad[:, :HF]
    x1 = x_pad[:, HF:]
    batch3 = jnp.pad(batch, (0, N_PAD - N), constant_values=G).reshape(
        N_BLKS, 1, ROW_BLK)

    zrow = jnp.zeros((K, HF), jnp.float32)
    z16 = jnp.zeros((ROWS_PER_TILE, 16), jnp.float32)
    ones16 = jnp.ones((K, 16), jnp.float32)

    fc2_Wp = jnp.zeros((F, F), jnp.float32).at[:C].set(fc2_W)
    fc2_bp = jnp.zeros((F,), jnp.float32).at[:C].set(fc2_b)

    sc = _make_sc_aggregate(m0, m1)

    h0, h1 = x0, x1
    for wl, bl, wr in ((Wl1, bl1, Wr1), (Wl2, bl2, Wr2), (Wl3, bl3, Wr3),
                       (Wl4, bl4, Wr4)):
        seg0, seg1, cnt = sc(h0, h1, src, dst, zrow, z16, ones16)
        h0, h1 = _tc_layer(seg0, seg1, cnt, h0, h1, wl, bl.reshape(1, F), wr)

    out = _tc_pool(h0, h1, batch3, fc1_W, fc1_b.reshape(1, F), fc2_Wp,
                   fc2_bp.reshape(1, F))
    return out[:, :C]


# speedup vs baseline: 1.0456x; 1.0456x over previous
import jax
import jax.numpy as jnp
from jax import lax
from jax.experimental import pallas as pl
from jax.experimental.pallas import tpu as pltpu
from jax.experimental.pallas import tpu_sc as plsc

N = 10000
F = 128
HF = 64
G = 64
C = 10

NC = 2
NS = 16
NW = NC * NS

N_PAD = 10240
ROWS_PER_TILE = N_PAD // NS
ABSORB = N

K = 128
NBUF = 3

_MESH = plsc.VectorSubcoreMesh(
    core_axis_name="c", subcore_axis_name="s", num_cores=NC, num_subcores=NS)


def _sc_body_seg(m0, m1, args):
    (x0_hbm, x1_hbm, src_hbm, dst_hbm, zrow_hbm, z16_hbm, ones_hbm,
     seg0_hbm, seg1_hbm, cnt_hbm, src_all, dst_all, g0, g1, g2,
     sg0, sg1, sg2, ss0, ss1, ss2,
     accum, onesv, c16v, cntacc, sem_c) = args
    gbufs = (g0, g1, g2)
    sem_g = (sg0, sg1, sg2)
    sem_s = (ss0, ss1, ss2)

    cid = lax.axis_index("c")
    sid = lax.axis_index("s")
    row0 = sid * ROWS_PER_TILE
    rows_sl = pl.ds(row0, ROWS_PER_TILE)

    m = jnp.where(cid == 0, m0, m1)
    base = jnp.where(cid == 0, sid * m0, NS * m0 + sid * m1)

    pltpu.sync_copy(src_hbm.at[pl.ds(base, m1)],
                    src_all.at[pl.ds(0, m1)])
    pltpu.sync_copy(dst_hbm.at[pl.ds(base, m1)],
                    dst_all.at[pl.ds(0, m1)])

    @pl.when(cid == 0)
    def _():
        pltpu.sync_copy(src_hbm.at[pl.ds(base + m1, m0 - m1)],
                        src_all.at[pl.ds(m1, m0 - m1)])
        pltpu.sync_copy(dst_hbm.at[pl.ds(base + m1, m0 - m1)],
                        dst_all.at[pl.ds(m1, m0 - m1)])

    for p, (x_hbm, seg_hbm) in enumerate(((x0_hbm, seg0_hbm),
                                          (x1_hbm, seg1_hbm))):
        pltpu.sync_copy(zrow_hbm, g1)
        for z in range(ROWS_PER_TILE // K):
            pltpu.async_copy(g1, accum.at[pl.ds(row0 + z * K, K)],
                             sem_s[z % NBUF])
        if p == 0:
            pltpu.sync_copy(ones_hbm, onesv)
            pltpu.sync_copy(z16_hbm, c16v)
            pltpu.async_copy(c16v, cntacc.at[rows_sl], sem_c)
        for z in range(ROWS_PER_TILE // K):
            pltpu.make_async_copy(g1, accum.at[pl.ds(row0 + z * K, K)],
                                  sem_s[z % NBUF]).wait()
        if p == 0:
            pltpu.make_async_copy(c16v, cntacc.at[rows_sl], sem_c).wait()
        plsc.subcore_barrier()

        for b in range(NBUF - 1):
            pltpu.async_copy(x_hbm.at[src_all.at[b]], gbufs[b], sem_g[b])

        def iter_body(j, carry):
            for bb in range(NBUF):
                @pl.when(j % NBUF == bb)
                def _():
                    b2 = (bb + NBUF - 1) % NBUF
                    pltpu.make_async_copy(
                        x_hbm.at[src_all.at[0]], gbufs[bb], sem_g[bb]).wait()
                    pltpu.async_copy(gbufs[bb], accum.at[dst_all.at[j]],
                                     sem_s[bb], add=True)
                    if p == 0:
                        @pl.when(j >= 1)
                        def _():
                            pltpu.make_async_copy(
                                onesv, cntacc.at[dst_all.at[0]], sem_c).wait()
                        pltpu.async_copy(onesv, cntacc.at[dst_all.at[j]],
                                         sem_c, add=True)

                    @pl.when(j + NBUF - 1 < m)
                    def _():
                        @pl.when(j >= 1)
                        def _():
                            pltpu.make_async_copy(
                                gbufs[b2], accum.at[dst_all.at[0]],
                                sem_s[b2]).wait()
                        pltpu.async_copy(x_hbm.at[src_all.at[j + NBUF - 1]],
                                         gbufs[b2], sem_g[b2])
            return carry

        lax.fori_loop(0, m, iter_body, 0)
        for bb in range(NBUF):
            pltpu.make_async_copy(gbufs[bb], accum.at[dst_all.at[0]],
                                  sem_s[bb]).wait()
        if p == 0:
            pltpu.make_async_copy(onesv, cntacc.at[dst_all.at[0]],
                                  sem_c).wait()
        plsc.subcore_barrier()

        nz = ROWS_PER_TILE // K
        sls = [pl.ds(row0 + z * K, K) for z in range(nz)]
        if p == 0:
            pltpu.async_copy(cntacc.at[rows_sl], c16v, sem_c)
        for z in range(min(NBUF, nz)):
            pltpu.async_copy(accum.at[sls[z]], gbufs[z], sem_g[z])
        for z in range(nz):
            b = z % NBUF
            pltpu.make_async_copy(accum.at[sls[z]], gbufs[b],
                                  sem_g[b]).wait()
            pltpu.async_copy(gbufs[b], seg_hbm.at[cid].at[sls[z]], sem_s[b])
            if z + NBUF < nz:
                pltpu.make_async_copy(gbufs[b], seg_hbm.at[cid].at[sls[z]],
                                      sem_s[b]).wait()
                pltpu.async_copy(accum.at[sls[z + NBUF]], gbufs[b], sem_g[b])
        for z in range(max(0, nz - NBUF), nz):
            b = z % NBUF
            pltpu.make_async_copy(gbufs[b], seg_hbm.at[cid].at[sls[z]],
                                  sem_s[b]).wait()
        if p == 0:
            pltpu.make_async_copy(cntacc.at[rows_sl], c16v, sem_c).wait()
            pltpu.sync_copy(c16v, cnt_hbm.at[cid].at[rows_sl])
        plsc.subcore_barrier()


def _make_sc_aggregate(m0, m1):
    seg_t = jax.ShapeDtypeStruct((NC, N_PAD, HF), jnp.float32)
    out_type = (seg_t, seg_t, jax.ShapeDtypeStruct((NC, N_PAD, 16),
                                                   jnp.float32))
    scratch = [
        pltpu.VMEM((m0, K), jnp.int32),
        pltpu.VMEM((m0, K), jnp.int32),
        pltpu.VMEM((K, HF), jnp.float32),
        pltpu.VMEM((K, HF), jnp.float32),
        pltpu.VMEM((K, HF), jnp.float32),
        pltpu.SemaphoreType.DMA,
        pltpu.SemaphoreType.DMA,
        pltpu.SemaphoreType.DMA,
        pltpu.SemaphoreType.DMA,
        pltpu.SemaphoreType.DMA,
        pltpu.SemaphoreType.DMA,
        pltpu.VMEM_SHARED((N_PAD, HF), jnp.float32),
        pltpu.VMEM((K, 16), jnp.float32),
        pltpu.VMEM((ROWS_PER_TILE, 16), jnp.float32),
        pltpu.VMEM_SHARED((N_PAD, 16), jnp.float32),
        pltpu.SemaphoreType.DMA,
    ]

    def body(*args):
        _sc_body_seg(m0, m1, args)

    return pl.kernel(body, out_type=out_type, mesh=_MESH,
                     scratch_types=tuple(scratch),
                     compiler_params=pltpu.CompilerParams(
                         use_tc_tiling_on_sc=False))


ROW_BLK = 256
N_BLKS = N_PAD // ROW_BLK


def _tc_layer_body(seg0_ref, seg1_ref, cnt_ref, x0_ref, x1_ref, wl_ref,
                   bl_ref, wr_ref, o0_ref, o1_ref):
    seg = jnp.concatenate([seg0_ref[0] + seg0_ref[1],
                           seg1_ref[0] + seg1_ref[1]], axis=1)
    x = jnp.concatenate([x0_ref[...], x1_ref[...]], axis=1)
    cnt = jnp.sum(cnt_ref[0] + cnt_ref[1], axis=1, keepdims=True) * (1.0 / 16.0)
    mean = seg / jnp.maximum(cnt, 1.0)
    dn = (((1,), (1,)), ((), ()))
    h = (lax.dot_general(mean, wl_ref[...], dn,
                         preferred_element_type=jnp.float32)
         + bl_ref[0]
         + lax.dot_general(x, wr_ref[...], dn,
                           preferred_element_type=jnp.float32))
    h = jnp.maximum(h, 0.0)
    o0_ref[...] = h[:, :HF]
    o1_ref[...] = h[:, HF:]


_half_t = jax.ShapeDtypeStruct((N_PAD, HF), jnp.float32)

_tc_layer = pl.pallas_call(
    _tc_layer_body,
    grid=(N_BLKS,),
    in_specs=[
        pl.BlockSpec((NC, ROW_BLK, HF), lambda i: (0, i, 0)),
        pl.BlockSpec((NC, ROW_BLK, HF), lambda i: (0, i, 0)),
        pl.BlockSpec((NC, ROW_BLK, 16), lambda i: (0, i, 0)),
        pl.BlockSpec((ROW_BLK, HF), lambda i: (i, 0)),
        pl.BlockSpec((ROW_BLK, HF), lambda i: (i, 0)),
        pl.BlockSpec((F, F), lambda i: (0, 0)),
        pl.BlockSpec((1, F), lambda i: (0, 0)),
        pl.BlockSpec((F, F), lambda i: (0, 0)),
    ],
    out_specs=[pl.BlockSpec((ROW_BLK, HF), lambda i: (i, 0)),
               pl.BlockSpec((ROW_BLK, HF), lambda i: (i, 0))],
    out_shape=[_half_t, _half_t],
)


def _tc_pool_body(h0_ref, h1_ref, b_ref, fc1w_ref, fc1b_ref, fc2w_ref,
                  fc2b_ref, o_ref, sum_scr, cnt_scr):
    i = pl.program_id(0)
    bids = b_ref[0, 0, :]
    h = jnp.concatenate([h0_ref[...], h1_ref[...]], axis=1)
    onehot = (lax.broadcasted_iota(jnp.int32, (G, ROW_BLK), 0)
              == bids[None, :]).astype(jnp.float32)
    psum = jnp.dot(onehot, h, preferred_element_type=jnp.float32)
    pcnt = jnp.broadcast_to(jnp.sum(onehot, axis=1, keepdims=True), (G, F))

    @pl.when(i == 0)
    def _():
        sum_scr[...] = jnp.zeros_like(sum_scr)
        cnt_scr[...] = jnp.zeros_like(cnt_scr)

    sum_scr[...] += psum
    cnt_scr[...] += pcnt

    @pl.when(i == N_BLKS - 1)
    def _():
        pooled = sum_scr[...] / jnp.maximum(cnt_scr[...], 1.0)
        dn = (((1,), (1,)), ((), ()))
        emb = lax.dot_general(pooled, fc1w_ref[...], dn,
                              preferred_element_type=jnp.float32) + fc1b_ref[0]
        o_ref[...] = lax.dot_general(emb, fc2w_ref[...], dn,
                                     preferred_element_type=jnp.float32) + fc2b_ref[0]


_tc_pool = pl.pallas_call(
    _tc_pool_body,
    grid=(N_BLKS,),
    in_specs=[
        pl.BlockSpec((ROW_BLK, HF), lambda i: (i, 0)),
        pl.BlockSpec((ROW_BLK, HF), lambda i: (i, 0)),
        pl.BlockSpec((1, 1, ROW_BLK), lambda i: (i, 0, 0)),
        pl.BlockSpec((F, F), lambda i: (0, 0)),
        pl.BlockSpec((1, F), lambda i: (0, 0)),
        pl.BlockSpec((F, F), lambda i: (0, 0)),
        pl.BlockSpec((1, F), lambda i: (0, 0)),
    ],
    out_specs=pl.BlockSpec((G, F), lambda i: (0, 0)),
    out_shape=jax.ShapeDtypeStruct((G, F), jnp.float32),
    scratch_shapes=[pltpu.VMEM((G, F), jnp.float32),
                    pltpu.VMEM((G, F), jnp.float32)],
)


def kernel(x, edge_index, batch, num_graphs, Wl1, bl1, Wr1, Wl2, bl2, Wr2,
           Wl3, bl3, Wr3, Wl4, bl4, Wr4, fc1_W, fc1_b, fc2_W, fc2_b):
    E = edge_index.shape[1]
    ct = -(-E // K)
    frac = 0.96
    m0 = max(NBUF, int(frac * ct / NS) // NBUF * NBUF)
    m1 = max(NBUF, -(-max(0, ct - NS * m0) // (NS * NBUF)) * NBUF)
    rows = NS * (m0 + m1)

    src = jnp.pad(edge_index[0], (0, rows * K - E)).reshape(rows, K)
    pad_dst = ABSORB + (jnp.arange(rows * K - E, dtype=jnp.int32)
                        % (N_PAD - N))
    dst = jnp.concatenate([edge_index[1], pad_dst]).reshape(rows, K)

    x_pad = jnp.pad(x, ((0, N_PAD - N), (0, 0)))
    x0 = x_pad[:, :HF]
    x1 = x_pad[:, HF:]
    batch3 = jnp.pad(batch, (0, N_PAD - N), constant_values=G).reshape(
        N_BLKS, 1, ROW_BLK)

    zrow = jnp.zeros((K, HF), jnp.float32)
    z16 = jnp.zeros((ROWS_PER_TILE, 16), jnp.float32)
    ones16 = jnp.ones((K, 16), jnp.float32)

    fc2_Wp = jnp.zeros((F, F), jnp.float32).at[:C].set(fc2_W)
    fc2_bp = jnp.zeros((F,), jnp.float32).at[:C].set(fc2_b)

    sc = _make_sc_aggregate(m0, m1)

    h0, h1 = x0, x1
    for wl, bl, wr in ((Wl1, bl1, Wr1), (Wl2, bl2, Wr2), (Wl3, bl3, Wr3),
                       (Wl4, bl4, Wr4)):
        seg0, seg1, cnt = sc(h0, h1, src, dst, zrow, z16, ones16)
        h0, h1 = _tc_layer(seg0, seg1, cnt, h0, h1, wl, bl.reshape(1, F), wr)

    out = _tc_pool(h0, h1, batch3, fc1_W, fc1_b.reshape(1, F), fc2_Wp,
                   fc2_bp.reshape(1, F))
    return out[:, :C]
